# baseline (device time: 32921 ns/iter reference)
import jax
import jax.numpy as jnp
from jax import lax
from jax.experimental import pallas as pl
from jax.experimental.pallas import tpu as pltpu

N_DEV = 4


def kernel(A, B):
    m_per, k = A.shape
    k2, n = B.shape
    assert k == k2
    m_half = m_per // 2
    TOP = pl.ds(0, m_half)
    BOT = pl.ds(m_half, m_half)

    def body(
        a_ref, b_ref, out_ref,
        my_q, my_s, b_bf, recv_lq, recv_ls, recv_rq, recv_rs,
        recv_dq, recv_ds, out_vmem,
        send_sems, recv_sems, copy_sems,
    ):
        my_pos = lax.axis_index("i")
        left = (my_pos - 1) % N_DEV
        right = (my_pos + 1) % N_DEV

        barrier_sem = pltpu.get_barrier_semaphore()
        for nbr in [left, right]:
            pl.semaphore_signal(
                barrier_sem, inc=1,
                device_id=(nbr,), device_id_type=pl.DeviceIdType.MESH,
            )
        a = a_ref[:, :]
        amax = jnp.maximum(
            jnp.max(jnp.abs(a), axis=1, keepdims=True), 1e-20
        )
        my_s[:, :] = amax * (1.0 / 127.0)
        my_q[:, :] = jnp.clip(
            jnp.round(a * (127.0 / amax)), -127.0, 127.0
        ).astype(jnp.int8)
        pl.semaphore_wait(barrier_sem, 2)

        def rdma(i, src, dst, dev):
            return pltpu.make_async_remote_copy(
                src_ref=src, dst_ref=dst,
                send_sem=send_sems.at[i], recv_sem=recv_sems.at[i],
                device_id=(dev,), device_id_type=pl.DeviceIdType.MESH,
            )

        p1 = [
            rdma(0, my_s.at[TOP, :], recv_ls.at[TOP, :], right),
            rdma(1, my_q.at[TOP, :], recv_lq.at[TOP, :], right),
            rdma(2, my_s.at[BOT, :], recv_rs.at[BOT, :], left),
            rdma(3, my_q.at[BOT, :], recv_rq.at[BOT, :], left),
            rdma(4, my_s.at[BOT, :], recv_ls.at[BOT, :], right),
            rdma(5, my_q.at[BOT, :], recv_lq.at[BOT, :], right),
            rdma(6, my_s.at[TOP, :], recv_rs.at[TOP, :], left),
            rdma(7, my_q.at[TOP, :], recv_rq.at[TOP, :], left),
        ]
        for r in p1:
            r.start()
        (s_rt_s, s_rt_q, s_lb_s, s_lb_q, s_rb_s, s_rb_q, s_lt_s, s_lt_q) = p1

        b_bf[:, :] = b_ref[:, :].astype(jnp.bfloat16)

        def mm_q(slot, rows, q_ref, s_ref):
            deq = (
                q_ref[rows, :].astype(jnp.float32) * s_ref[rows, :]
            ).astype(jnp.bfloat16)
            out_vmem[slot, rows, :] = jnp.dot(
                deq, b_bf[:, :], preferred_element_type=jnp.float32
            )

        def store_half(sem_i, slot, rows, origin, row_off):
            copy = pltpu.make_async_copy(
                out_vmem.at[slot, rows, :],
                out_ref.at[pl.ds(origin * m_per + row_off, m_half), :],
                copy_sems.at[sem_i],
            )
            copy.start()
            return copy

        a_bf_top = a_ref[TOP, :].astype(jnp.bfloat16)
        out_vmem[0, TOP, :] = jnp.dot(
            a_bf_top, b_bf[:, :], preferred_element_type=jnp.float32
        )
        c0 = store_half(0, 0, TOP, my_pos, 0)
        a_bf_bot = a_ref[BOT, :].astype(jnp.bfloat16)
        out_vmem[0, BOT, :] = jnp.dot(
            a_bf_bot, b_bf[:, :], preferred_element_type=jnp.float32
        )
        c1 = store_half(1, 0, BOT, my_pos, m_half)

        s_rt_s.wait_recv()
        s_rt_q.wait_recv()
        f_r_s = rdma(8, recv_ls.at[TOP, :], recv_ds.at[TOP, :], right)
        f_r_q = rdma(9, recv_lq.at[TOP, :], recv_dq.at[TOP, :], right)
        f_r_s.start()
        f_r_q.start()
        mm_q(1, TOP, recv_lq, recv_ls)
        c2 = store_half(2, 1, TOP, left, 0)

        s_lb_s.wait_recv()
        s_lb_q.wait_recv()
        f_l_s = rdma(10, recv_rs.at[BOT, :], recv_ds.at[BOT, :], left)
        f_l_q = rdma(11, recv_rq.at[BOT, :], recv_dq.at[BOT, :], left)
        f_l_s.start()
        f_l_q.start()
        mm_q(2, BOT, recv_rq, recv_rs)
        c3 = store_half(3, 2, BOT, right, m_half)

        s_rb_s.wait_recv()
        s_rb_q.wait_recv()
        mm_q(1, BOT, recv_lq, recv_ls)
        c4 = store_half(4, 1, BOT, left, m_half)
        s_lt_s.wait_recv()
        s_lt_q.wait_recv()
        mm_q(2, TOP, recv_rq, recv_rs)
        c5 = store_half(5, 2, TOP, right, 0)

        diag = (my_pos + 2) % N_DEV
        f_r_s.wait_recv()
        f_r_q.wait_recv()
        mm_q(3, TOP, recv_dq, recv_ds)
        c6 = store_half(6, 3, TOP, diag, 0)
        f_l_s.wait_recv()
        f_l_q.wait_recv()
        mm_q(3, BOT, recv_dq, recv_ds)
        c7 = store_half(7, 3, BOT, diag, m_half)

        for c in [c0, c1, c2, c3, c4, c5, c6, c7]:
            c.wait()
        for s in p1 + [f_r_s, f_r_q, f_l_s, f_l_q]:
            s.wait_send()

    return pl.pallas_call(
        body,
        out_shape=jax.ShapeDtypeStruct((N_DEV * m_per, n), jnp.float32),
        in_specs=[
            pl.BlockSpec(memory_space=pltpu.VMEM),
            pl.BlockSpec(memory_space=pltpu.VMEM),
        ],
        out_specs=pl.BlockSpec(memory_space=pl.ANY),
        scratch_shapes=[
            pltpu.VMEM((m_per, k), jnp.int8),
            pltpu.VMEM((m_per, 1), jnp.float32),
            pltpu.VMEM((k, n), jnp.bfloat16),
            pltpu.VMEM((m_per, k), jnp.int8),
            pltpu.VMEM((m_per, 1), jnp.float32),
            pltpu.VMEM((m_per, k), jnp.int8),
            pltpu.VMEM((m_per, 1), jnp.float32),
            pltpu.VMEM((m_per, k), jnp.int8),
            pltpu.VMEM((m_per, 1), jnp.float32),
            pltpu.VMEM((N_DEV, m_per, n), jnp.float32),
            pltpu.SemaphoreType.DMA((12,)),
            pltpu.SemaphoreType.DMA((12,)),
            pltpu.SemaphoreType.DMA((8,)),
        ],
        compiler_params=pltpu.CompilerParams(collective_id=0),
    )(A, B)
